# baseline (speedup 1.0000x reference)
import functools

import jax
import jax.numpy as jnp
from jax import lax
from jax.experimental import pallas as pl
from jax.experimental.pallas import tpu as pltpu
from jax.experimental.pallas import tpu_sc as plsc

_DIM = 64
_N_EMBED = 1024
_ROWS = 16
_COLS = 1024
_TOKENS = _ROWS * _COLS
_BR = 2
_BLK = _BR * _COLS
_NUM_CHUNKS = _ROWS // _BR


def _vq_body(x_ref, e_ref, ind_ref, diff_ref, ppl_ref, et_ref):
    e = e_ref[...]
    et_ref[...] = e.T
    e_sq = jnp.sum(e * e, axis=0, keepdims=True)
    iota = jax.lax.broadcasted_iota(jnp.int32, (_BLK, _N_EMBED), 1)

    cnt = jnp.zeros((_N_EMBED,), dtype=jnp.float32)
    dsum = jnp.float32(0.0)
    for c in range(_NUM_CHUNKS):
        x = x_ref[c * _BR:(c + 1) * _BR].reshape(_BLK, _DIM)
        neg2_scores = jax.lax.dot_general(
            x * (-2.0), e, (((1,), (0,)), ((), ())),
            preferred_element_type=jnp.float32)
        x_sq = jnp.sum(x * x, axis=1, keepdims=True)
        dist = (x_sq + neg2_scores) + e_sq

        ind = jnp.argmin(dist, axis=1).astype(jnp.int32)
        onehot = (iota == ind[:, None]).astype(jnp.float32)

        ind_ref[c * _BLK:(c + 1) * _BLK] = ind

        ones = jnp.ones((1, _BLK), dtype=jnp.float32)
        cnt = cnt + jax.lax.dot_general(
            ones, onehot, (((1,), (0,)), ((), ())),
            preferred_element_type=jnp.float32)[0]
        dsum = dsum + jnp.sum(jnp.min(dist, axis=1))

    diff_ref[...] = jnp.reshape(dsum / float(_TOKENS * _DIM), (1, 1))
    p = cnt / float(_TOKENS)
    ent = jnp.sum(p * jnp.log(jnp.clip(p, 1e-7, None)), keepdims=True)
    ppl_ref[...] = jnp.exp(-ent).reshape(1, 1)


_SC_NUM_CORES = 2
_SC_NUM_SUBCORES = 16
_NW = _SC_NUM_CORES * _SC_NUM_SUBCORES
_B_PER_W = _TOKENS // _NW


@functools.lru_cache(maxsize=1)
def _build_sc_gather():
    @functools.partial(
        pl.kernel,
        mesh=plsc.VectorSubcoreMesh(
            core_axis_name="c", subcore_axis_name="s",
            num_cores=_SC_NUM_CORES, num_subcores=_SC_NUM_SUBCORES),
        out_type=jax.ShapeDtypeStruct((_TOKENS, _DIM), jnp.float32),
        scratch_types=[
            pltpu.VMEM((_B_PER_W,), jnp.int32),
            pltpu.VMEM((_B_PER_W, _DIM), jnp.float32),
            pltpu.SemaphoreType.DMA,
        ],
        compiler_params=pltpu.CompilerParams(use_tc_tiling_on_sc=False),
    )
    def _sc_gather(table_hbm, idx_hbm, out_hbm, idx_v, rows_v, sem):
        wid = lax.axis_index("s") * _SC_NUM_CORES + lax.axis_index("c")
        base = wid * _B_PER_W
        pltpu.sync_copy(idx_hbm.at[pl.ds(base, _B_PER_W)], idx_v)
        pltpu.async_copy(table_hbm.at[idx_v], rows_v, sem).wait()
        pltpu.sync_copy(rows_v, out_hbm.at[pl.ds(base, _B_PER_W)])

    return _sc_gather


@functools.partial(jax.jit, static_argnames=())
def kernel(input, embed):
    ind, diff, ppl, e_t = pl.pallas_call(
        _vq_body,
        out_shape=[
            jax.ShapeDtypeStruct((_TOKENS,), jnp.int32),
            jax.ShapeDtypeStruct((1, 1), jnp.float32),
            jax.ShapeDtypeStruct((1, 1), jnp.float32),
            jax.ShapeDtypeStruct((_N_EMBED, _DIM), jnp.float32),
        ],
    )(input, embed)
    q = _build_sc_gather()(e_t, ind)
    return (q.reshape(_ROWS, _COLS, _DIM), diff[0, 0],
            ind.reshape(_ROWS, _COLS), ppl[0, 0])


# speedup vs baseline: 1.9204x; 1.9204x over previous
import functools

import jax
import jax.numpy as jnp
from jax.experimental import pallas as pl
from jax.experimental.pallas import tpu as pltpu

_DIM = 64
_N_EMBED = 1024
_ROWS = 16
_COLS = 1024
_TOKENS = _ROWS * _COLS
_BR = 4
_BLK = _BR * _COLS
_NUM_CHUNKS = _ROWS // _BR


def _vq_body(x_ref, e_ref, q_ref, ind_ref, diff_ref, ppl_ref):
    e = e_ref[...]
    e_sq = jnp.sum(e * e, axis=0, keepdims=True)
    iota = jax.lax.broadcasted_iota(jnp.int32, (_BLK, _N_EMBED), 1)

    cnt = jnp.zeros((_N_EMBED,), dtype=jnp.float32)
    dsum = jnp.float32(0.0)
    for c in range(_NUM_CHUNKS):
        x = x_ref[c * _BR:(c + 1) * _BR].reshape(_BLK, _DIM)
        neg2_scores = jax.lax.dot_general(
            x * (-2.0), e, (((1,), (0,)), ((), ())),
            preferred_element_type=jnp.float32)
        x_sq = jnp.sum(x * x, axis=1, keepdims=True)
        dist = (x_sq + neg2_scores) + e_sq

        ind = jnp.argmin(dist, axis=1).astype(jnp.int32)
        onehot = (iota == ind[:, None]).astype(jnp.float32)
        q = jax.lax.dot_general(
            onehot, e, (((1,), (1,)), ((), ())),
            preferred_element_type=jnp.float32)

        q_ref[c * _BR:(c + 1) * _BR] = (x + (q - x)).reshape(_BR, _COLS, _DIM)
        ind_ref[c * _BLK:(c + 1) * _BLK] = ind

        ones = jnp.ones((1, _BLK), dtype=jnp.float32)
        cnt = cnt + jax.lax.dot_general(
            ones, onehot, (((1,), (0,)), ((), ())),
            preferred_element_type=jnp.float32)[0]
        dsum = dsum + jnp.sum((q - x) ** 2)

    diff_ref[...] = jnp.reshape(dsum / float(_TOKENS * _DIM), (1, 1))
    p = cnt / float(_TOKENS)
    ent = jnp.sum(p * jnp.log(jnp.clip(p, 1e-7, None)), keepdims=True)
    ppl_ref[...] = jnp.exp(-ent).reshape(1, 1)


@functools.partial(jax.jit, static_argnames=())
def kernel(input, embed):
    q, ind, diff, ppl = pl.pallas_call(
        _vq_body,
        out_shape=[
            jax.ShapeDtypeStruct((_ROWS, _COLS, _DIM), jnp.float32),
            jax.ShapeDtypeStruct((_TOKENS,), jnp.int32),
            jax.ShapeDtypeStruct((1, 1), jnp.float32),
            jax.ShapeDtypeStruct((1, 1), jnp.float32),
        ],
    )(input, embed)
    return q, diff[0, 0], ind.reshape(_ROWS, _COLS), ppl[0, 0]


# speedup vs baseline: 2.1629x; 1.1263x over previous
import functools

import jax
import jax.numpy as jnp
from jax.experimental import pallas as pl
from jax.experimental.pallas import tpu as pltpu

_DIM = 64
_N_EMBED = 1024
_ROWS = 16
_COLS = 1024
_TOKENS = _ROWS * _COLS
_BR = 1
_BLK = _BR * _COLS
_NUM_CHUNKS = _ROWS // _BR


def _vq_body(x_ref, e_ref, q_ref, ind_ref, diff_ref, ppl_ref):
    e = e_ref[...]
    e_sq = jnp.sum(e * e, axis=0, keepdims=True)
    iota = jax.lax.broadcasted_iota(jnp.int32, (_BLK, _N_EMBED), 1)

    cnt = jnp.zeros((_N_EMBED,), dtype=jnp.float32)
    dsum = jnp.float32(0.0)
    for c in range(_NUM_CHUNKS):
        x = x_ref[c * _BR:(c + 1) * _BR].reshape(_BLK, _DIM)
        neg2_scores = jax.lax.dot_general(
            x * (-2.0), e, (((1,), (0,)), ((), ())),
            preferred_element_type=jnp.float32)
        x_sq = jnp.sum(x * x, axis=1, keepdims=True)
        dist = (x_sq + neg2_scores) + e_sq

        ind = jnp.argmin(dist, axis=1).astype(jnp.int32)
        onehot = (iota == ind[:, None]).astype(jnp.float32)
        q = jax.lax.dot_general(
            onehot, e, (((1,), (1,)), ((), ())),
            preferred_element_type=jnp.float32)

        q_ref[c * _BR:(c + 1) * _BR] = (x + (q - x)).reshape(_BR, _COLS, _DIM)
        ind_ref[c * _BLK:(c + 1) * _BLK] = ind

        ones = jnp.ones((1, _BLK), dtype=jnp.float32)
        cnt = cnt + jax.lax.dot_general(
            ones, onehot, (((1,), (0,)), ((), ())),
            preferred_element_type=jnp.float32)[0]
        dsum = dsum + jnp.sum((q - x) ** 2)

    diff_ref[...] = jnp.reshape(dsum / float(_TOKENS * _DIM), (1, 1))
    p = cnt / float(_TOKENS)
    ent = jnp.sum(p * jnp.log(jnp.clip(p, 1e-7, None)), keepdims=True)
    ppl_ref[...] = jnp.exp(-ent).reshape(1, 1)


@functools.partial(jax.jit, static_argnames=())
def kernel(input, embed):
    q, ind, diff, ppl = pl.pallas_call(
        _vq_body,
        out_shape=[
            jax.ShapeDtypeStruct((_ROWS, _COLS, _DIM), jnp.float32),
            jax.ShapeDtypeStruct((_TOKENS,), jnp.int32),
            jax.ShapeDtypeStruct((1, 1), jnp.float32),
            jax.ShapeDtypeStruct((1, 1), jnp.float32),
        ],
    )(input, embed)
    return q, diff[0, 0], ind.reshape(_ROWS, _COLS), ppl[0, 0]
